# baseline (speedup 1.0000x reference)
import functools

import jax
import jax.numpy as jnp
from jax.experimental import pallas as pl
from jax.experimental.pallas import tpu as pltpu

_L = 2048
_QT = 128


def _dist_body(xc_ref, xr_ref, out_ref):
    dx = xc_ref[0, 0] - xr_ref[0, 0]
    dy = xc_ref[0, 1] - xr_ref[0, 1]
    dz = xc_ref[0, 2] - xr_ref[0, 2]
    s = dx * dx + dy * dy
    s = s + dz * dz
    s = s + jnp.float32(1e-8)
    out_ref[0] = jnp.sqrt(s)


def kernel(X, coord_mask, padding_mask, top_k_neighbors):
    del coord_mask, padding_mask, top_k_neighbors
    bsz, maxlen = X.shape[0], X.shape[1]
    xt = jnp.transpose(X, (0, 2, 1))
    xc = xt[:, :, :, None]
    xr = xt[:, :, None, :]
    grid = (bsz, maxlen // _QT)
    D = pl.pallas_call(
        _dist_body,
        grid=grid,
        in_specs=[
            pl.BlockSpec((1, 3, _QT, 1), lambda b, i: (b, 0, i, 0)),
            pl.BlockSpec((1, 3, 1, maxlen), lambda b, i: (b, 0, 0, 0)),
        ],
        out_specs=pl.BlockSpec((1, _QT, maxlen), lambda b, i: (b, i, 0)),
        out_shape=jax.ShapeDtypeStruct((bsz, maxlen, maxlen), jnp.float32),
    )(xc, xr)
    neg_vals, E_idx = jax.lax.top_k(-D, 32)
    D_neighbors = -neg_vals
    coord_mask_neighbors = D_neighbors < 5e7
    residue_mask_neighbors = D_neighbors < 5e9
    return (D_neighbors, E_idx, coord_mask_neighbors, residue_mask_neighbors)


# speedup vs baseline: 1.2224x; 1.2224x over previous
import functools

import numpy as np

import jax
import jax.numpy as jnp
from jax import lax
from jax.experimental import pallas as pl
from jax.experimental.pallas import tpu as pltpu
from jax.experimental.pallas import tpu_sc as plsc

_NC, _NS, _LANES = 2, 16, 16
_NW = _NC * _NS
_KC = 48
_NLVL = _KC // _LANES
_K = 32
_NPASS = 10
_EPS = np.float32(1e-8)


def _sc_select(xt, B, L):
    rows = B * L
    qpw = rows // _NW
    nchunk = L // _LANES

    def body(xt_hbm, cs_hbm, ci_hbm, xk, outs, outi, pend_s, pend_i, tv, cnt):
        cid = lax.axis_index("c")
        sid = lax.axis_index("s")
        wid = sid * _NC + cid
        pltpu.sync_copy(xt_hbm, xk)
        qbase = wid * qpw
        inf16 = jnp.full((_LANES,), jnp.inf, jnp.float32)
        zero16 = jnp.zeros((_LANES,), jnp.int32)
        lanes_i = lax.iota(jnp.int32, _LANES)

        def merge16(q, ks, vs):
            for lvl in range(_NLVL):
                bk = outs[q, pl.ds(lvl * _LANES, _LANES)]
                bi = outi[q, pl.ds(lvl * _LANES, _LANES)]
                rk = lax.rev(ks, (0,))
                ri = lax.rev(vs, (0,))
                m = bk <= rk
                lok = jnp.where(m, bk, rk)
                loi = jnp.where(m, bi, ri)
                hik = jnp.where(m, rk, bk)
                hii = jnp.where(m, ri, bi)
                lok, loi = plsc.sort_key_val(lok, loi)
                outs[q, pl.ds(lvl * _LANES, _LANES)] = lok
                outi[q, pl.ds(lvl * _LANES, _LANES)] = loi
                if lvl < _NLVL - 1:
                    ks, vs = plsc.sort_key_val(hik, hii)
            tv[...] = jnp.full((_LANES,), jnp.max(lok), jnp.float32)

        def do_query(q, carry):
            row = qbase + q
            b = row // L
            i = row - b * L
            xoff = b * (3 * L) + i
            qx = plsc.load_gather(xk, [jnp.full((_LANES,), xoff, jnp.int32)])
            qy = plsc.load_gather(xk, [jnp.full((_LANES,), xoff + L,
                                                jnp.int32)])
            qz = plsc.load_gather(xk, [jnp.full((_LANES,), xoff + 2 * L,
                                                jnp.int32)])
            for lvl in range(_NLVL):
                outs[q, pl.ds(lvl * _LANES, _LANES)] = inf16
                outi[q, pl.ds(lvl * _LANES, _LANES)] = zero16
            tv[...] = inf16
            cnt[0] = 0

            kbase = b * (3 * L)

            def do_chunk(j, c2):
                base = j * _LANES
                kx = xk[pl.ds(kbase + base, _LANES)]
                ky = xk[pl.ds(kbase + L + base, _LANES)]
                kz = xk[pl.ds(kbase + 2 * L + base, _LANES)]
                dx = kx - qx
                dy = ky - qy
                dz = kz - qz
                s = ((dx * dx + dy * dy) + dz * dz) + _EPS
                m = s < tv[...]
                hit = jnp.any(m)

                @pl.when(hit)
                def _():
                    idxv = lanes_i + base
                    c0 = cnt[0]
                    plsc.store_compressed(pend_s.at[pl.ds(c0, _LANES)], s,
                                          mask=m)
                    plsc.store_compressed(pend_i.at[pl.ds(c0, _LANES)], idxv,
                                          mask=m)
                    c1 = c0 + jnp.sum(m.astype(jnp.int32))
                    cnt[0] = c1

                    @pl.when(c1 >= _LANES)
                    def _():
                        ps = pend_s[pl.ds(0, _LANES)]
                        pi = pend_i[pl.ds(0, _LANES)]
                        ks, vs = plsc.sort_key_val(ps, pi)
                        merge16(q, ks, vs)
                        pend_s[pl.ds(0, _LANES)] = pend_s[pl.ds(_LANES, _LANES)]
                        pend_i[pl.ds(0, _LANES)] = pend_i[pl.ds(_LANES, _LANES)]
                        cnt[0] = c1 - _LANES

                return c2

            lax.fori_loop(0, nchunk, do_chunk, 0)
            c = cnt[0]

            @pl.when(c > 0)
            def _():
                ps = pend_s[pl.ds(0, _LANES)]
                pi = pend_i[pl.ds(0, _LANES)]
                ps = jnp.where(lanes_i < c, ps, jnp.inf)
                ks, vs = plsc.sort_key_val(ps, pi)
                merge16(q, ks, vs)

            return carry

        lax.fori_loop(0, qpw, do_query, 0)
        pltpu.sync_copy(outs, cs_hbm.at[pl.ds(qbase, qpw)])
        pltpu.sync_copy(outi, ci_hbm.at[pl.ds(qbase, qpw)])

    mesh = plsc.VectorSubcoreMesh(core_axis_name="c", subcore_axis_name="s",
                                  num_cores=_NC, num_subcores=_NS)
    f = pl.kernel(
        body,
        out_type=[jax.ShapeDtypeStruct((rows, _KC), jnp.float32),
                  jax.ShapeDtypeStruct((rows, _KC), jnp.int32)],
        mesh=mesh,
        compiler_params=pltpu.CompilerParams(needs_layout_passes=False),
        scratch_types=[
            pltpu.VMEM((B * 3 * L,), jnp.float32),
            pltpu.VMEM((qpw, _KC), jnp.float32),
            pltpu.VMEM((qpw, _KC), jnp.int32),
            pltpu.VMEM((_KC,), jnp.float32),
            pltpu.VMEM((_KC,), jnp.int32),
            pltpu.VMEM((_LANES,), jnp.float32),
            pltpu.SMEM((1,), jnp.int32),
        ],
    )
    return f(xt)


def _fix_body(cs_ref, ci_ref, d_ref, i_ref, cm_ref, rm_ref):
    s = cs_ref[...]
    ix = ci_ref[...]
    d = jnp.sqrt(s)
    n = s.shape[-1]
    lane = lax.broadcasted_iota(jnp.int32, s.shape, 1)
    for p in range(_NPASS):
        par = p % 2
        enR = (lane % 2 == par) & (lane < n - 1)
        enL = ((lane + 1) % 2 == par) & (lane >= 1)
        dR = pltpu.roll(d, n - 1, 1)
        ixR = pltpu.roll(ix, n - 1, 1)
        dL = pltpu.roll(d, 1, 1)
        ixL = pltpu.roll(ix, 1, 1)
        swapR = (d == dR) & (ix > ixR) & enR
        swapL = (dL == d) & (ixL > ix) & enL
        ix = jnp.where(swapR, ixR, jnp.where(swapL, ixL, ix))
    d32 = d[:, :_K]
    i32 = ix[:, :_K]
    d_ref[...] = d32
    i_ref[...] = i32
    cm_ref[...] = d32 < jnp.float32(5e7)
    rm_ref[...] = d32 < jnp.float32(5e9)


def _tc_fix(cs, ci):
    rows = cs.shape[0]
    rb = min(512, rows)
    grid = (rows // rb,)
    return pl.pallas_call(
        _fix_body,
        grid=grid,
        in_specs=[
            pl.BlockSpec((rb, _KC), lambda r: (r, 0)),
            pl.BlockSpec((rb, _KC), lambda r: (r, 0)),
        ],
        out_specs=[
            pl.BlockSpec((rb, _K), lambda r: (r, 0)),
            pl.BlockSpec((rb, _K), lambda r: (r, 0)),
            pl.BlockSpec((rb, _K), lambda r: (r, 0)),
            pl.BlockSpec((rb, _K), lambda r: (r, 0)),
        ],
        out_shape=[
            jax.ShapeDtypeStruct((rows, _K), jnp.float32),
            jax.ShapeDtypeStruct((rows, _K), jnp.int32),
            jax.ShapeDtypeStruct((rows, _K), jnp.bool_),
            jax.ShapeDtypeStruct((rows, _K), jnp.bool_),
        ],
    )(cs, ci)


def kernel(X, coord_mask, padding_mask, top_k_neighbors):
    del coord_mask, padding_mask, top_k_neighbors
    B, L = X.shape[0], X.shape[1]
    xt = jnp.transpose(X, (0, 2, 1)).reshape(-1)
    cs, ci = _sc_select(xt, B, L)
    d32, i32, cm, rm = _tc_fix(cs, ci)
    shape = (B, L, _K)
    return (d32.reshape(shape), i32.reshape(shape),
            cm.reshape(shape), rm.reshape(shape))


# speedup vs baseline: 1.3389x; 1.0954x over previous
import functools

import numpy as np

import jax
import jax.numpy as jnp
from jax import lax
from jax.experimental import pallas as pl
from jax.experimental.pallas import tpu as pltpu
from jax.experimental.pallas import tpu_sc as plsc

_NC, _NS, _LANES = 2, 16, 16
_NW = _NC * _NS
_KC = 48
_NLVL = _KC // _LANES
_K = 32
_NPASS = 10
_CPB = 4
_EPS = np.float32(1e-8)


def _sc_select(xt, B, L):
    rows = B * L
    qpw = rows // _NW
    nchunk = L // _LANES

    def body(xt_hbm, cs_hbm, ci_hbm, xk, outs, outi, pend_s, pend_i, tv, cnt):
        cid = lax.axis_index("c")
        sid = lax.axis_index("s")
        wid = sid * _NC + cid
        pltpu.sync_copy(xt_hbm, xk)
        qbase = wid * qpw
        inf16 = jnp.full((_LANES,), jnp.inf, jnp.float32)
        zero16 = jnp.zeros((_LANES,), jnp.int32)
        lanes_i = lax.iota(jnp.int32, _LANES)

        def merge16(q, ks, vs):
            for lvl in range(_NLVL):
                bk = outs[q, pl.ds(lvl * _LANES, _LANES)]
                bi = outi[q, pl.ds(lvl * _LANES, _LANES)]
                rk = lax.rev(ks, (0,))
                ri = lax.rev(vs, (0,))
                m = bk <= rk
                lok = jnp.where(m, bk, rk)
                loi = jnp.where(m, bi, ri)
                hik = jnp.where(m, rk, bk)
                hii = jnp.where(m, ri, bi)
                lok, loi = plsc.sort_key_val(lok, loi)
                outs[q, pl.ds(lvl * _LANES, _LANES)] = lok
                outi[q, pl.ds(lvl * _LANES, _LANES)] = loi
                if lvl < _NLVL - 1:
                    ks, vs = plsc.sort_key_val(hik, hii)
            tv[...] = jnp.full((_LANES,), jnp.max(lok), jnp.float32)

        def do_query(q, carry):
            row = qbase + q
            b = row // L
            i = row - b * L
            xoff = b * (3 * L) + i
            qx = plsc.load_gather(xk, [jnp.full((_LANES,), xoff, jnp.int32)])
            qy = plsc.load_gather(xk, [jnp.full((_LANES,), xoff + L,
                                                jnp.int32)])
            qz = plsc.load_gather(xk, [jnp.full((_LANES,), xoff + 2 * L,
                                                jnp.int32)])
            for lvl in range(_NLVL):
                outs[q, pl.ds(lvl * _LANES, _LANES)] = inf16
                outi[q, pl.ds(lvl * _LANES, _LANES)] = zero16
            tv[...] = inf16
            cnt[0] = 0

            kbase = b * (3 * L)

            def drain():
                ps = pend_s[pl.ds(0, _LANES)]
                pi = pend_i[pl.ds(0, _LANES)]
                ks, vs = plsc.sort_key_val(ps, pi)
                merge16(q, ks, vs)
                pend_s[pl.ds(0, _LANES)] = pend_s[pl.ds(_LANES, _LANES)]
                pend_i[pl.ds(0, _LANES)] = pend_i[pl.ds(_LANES, _LANES)]

            def do_iter(it, c2):
                base = it * (_LANES * _CPB)
                tvec = tv[...]
                ss = []
                for u in range(_CPB):
                    b0 = base + u * _LANES
                    kx = xk[pl.ds(kbase + b0, _LANES)]
                    ky = xk[pl.ds(kbase + L + b0, _LANES)]
                    kz = xk[pl.ds(kbase + 2 * L + b0, _LANES)]
                    dx = kx - qx
                    dy = ky - qy
                    dz = kz - qz
                    ss.append(((dx * dx + dy * dy) + dz * dz) + _EPS)
                m_any = ss[0] < tvec
                for u in range(1, _CPB):
                    m_any = m_any | (ss[u] < tvec)
                hit = jnp.any(m_any)

                @pl.when(hit)
                def _():
                    for u in range(_CPB):
                        m = ss[u] < tvec
                        idxv = lanes_i + (base + u * _LANES)
                        c0 = cnt[0]
                        plsc.store_compressed(pend_s.at[pl.ds(c0, _LANES)],
                                              ss[u], mask=m)
                        plsc.store_compressed(pend_i.at[pl.ds(c0, _LANES)],
                                              idxv, mask=m)
                        c1 = c0 + jnp.sum(m.astype(jnp.int32))
                        cnt[0] = c1

                        @pl.when(c1 >= _LANES)
                        def _():
                            drain()
                            cnt[0] = c1 - _LANES

                return c2

            lax.fori_loop(0, nchunk // _CPB, do_iter, 0)
            c = cnt[0]

            @pl.when(c > 0)
            def _():
                ps = pend_s[pl.ds(0, _LANES)]
                pi = pend_i[pl.ds(0, _LANES)]
                ps = jnp.where(lanes_i < c, ps, jnp.inf)
                ks, vs = plsc.sort_key_val(ps, pi)
                merge16(q, ks, vs)

            return carry

        lax.fori_loop(0, qpw, do_query, 0)
        pltpu.sync_copy(outs, cs_hbm.at[pl.ds(qbase, qpw)])
        pltpu.sync_copy(outi, ci_hbm.at[pl.ds(qbase, qpw)])

    mesh = plsc.VectorSubcoreMesh(core_axis_name="c", subcore_axis_name="s",
                                  num_cores=_NC, num_subcores=_NS)
    f = pl.kernel(
        body,
        out_type=[jax.ShapeDtypeStruct((rows, _KC), jnp.float32),
                  jax.ShapeDtypeStruct((rows, _KC), jnp.int32)],
        mesh=mesh,
        compiler_params=pltpu.CompilerParams(needs_layout_passes=False),
        scratch_types=[
            pltpu.VMEM((B * 3 * L,), jnp.float32),
            pltpu.VMEM((qpw, _KC), jnp.float32),
            pltpu.VMEM((qpw, _KC), jnp.int32),
            pltpu.VMEM((_KC,), jnp.float32),
            pltpu.VMEM((_KC,), jnp.int32),
            pltpu.VMEM((_LANES,), jnp.float32),
            pltpu.SMEM((1,), jnp.int32),
        ],
    )
    return f(xt)


def _fix_body(cs_ref, ci_ref, d_ref, i_ref, cm_ref, rm_ref):
    s = cs_ref[...]
    ix = ci_ref[...]
    d = jnp.sqrt(s)
    n = s.shape[-1]
    lane = lax.broadcasted_iota(jnp.int32, s.shape, 1)
    for p in range(_NPASS):
        par = p % 2
        enR = (lane % 2 == par) & (lane < n - 1)
        enL = ((lane + 1) % 2 == par) & (lane >= 1)
        dR = pltpu.roll(d, n - 1, 1)
        ixR = pltpu.roll(ix, n - 1, 1)
        dL = pltpu.roll(d, 1, 1)
        ixL = pltpu.roll(ix, 1, 1)
        swapR = (d == dR) & (ix > ixR) & enR
        swapL = (dL == d) & (ixL > ix) & enL
        ix = jnp.where(swapR, ixR, jnp.where(swapL, ixL, ix))
    d32 = d[:, :_K]
    i32 = ix[:, :_K]
    d_ref[...] = d32
    i_ref[...] = i32
    cm_ref[...] = d32 < jnp.float32(5e7)
    rm_ref[...] = d32 < jnp.float32(5e9)


def _tc_fix(cs, ci):
    rows = cs.shape[0]
    rb = min(512, rows)
    grid = (rows // rb,)
    return pl.pallas_call(
        _fix_body,
        grid=grid,
        in_specs=[
            pl.BlockSpec((rb, _KC), lambda r: (r, 0)),
            pl.BlockSpec((rb, _KC), lambda r: (r, 0)),
        ],
        out_specs=[
            pl.BlockSpec((rb, _K), lambda r: (r, 0)),
            pl.BlockSpec((rb, _K), lambda r: (r, 0)),
            pl.BlockSpec((rb, _K), lambda r: (r, 0)),
            pl.BlockSpec((rb, _K), lambda r: (r, 0)),
        ],
        out_shape=[
            jax.ShapeDtypeStruct((rows, _K), jnp.float32),
            jax.ShapeDtypeStruct((rows, _K), jnp.int32),
            jax.ShapeDtypeStruct((rows, _K), jnp.bool_),
            jax.ShapeDtypeStruct((rows, _K), jnp.bool_),
        ],
    )(cs, ci)


def kernel(X, coord_mask, padding_mask, top_k_neighbors):
    del coord_mask, padding_mask, top_k_neighbors
    B, L = X.shape[0], X.shape[1]
    xt = jnp.transpose(X, (0, 2, 1)).reshape(-1)
    cs, ci = _sc_select(xt, B, L)
    d32, i32, cm, rm = _tc_fix(cs, ci)
    shape = (B, L, _K)
    return (d32.reshape(shape), i32.reshape(shape),
            cm.reshape(shape), rm.reshape(shape))


# speedup vs baseline: 2.8090x; 2.0979x over previous
import functools

import numpy as np

import jax
import jax.numpy as jnp
from jax import lax
from jax.experimental import pallas as pl
from jax.experimental.pallas import tpu as pltpu
from jax.experimental.pallas import tpu_sc as plsc

_NC, _NS, _LANES = 2, 16, 16
_NW = _NC * _NS
_KC = 48
_NLVL = _KC // _LANES
_K = 32
_NPASS = 10
_CPB = 4
_EPS = np.float32(1e-8)


def _sc_select(xt, B, L):
    rows = B * L
    qpw = rows // _NW
    nchunk = L // _LANES

    def body(xt_hbm, cs_hbm, ci_hbm, xk, outs, outi, sbuf, ccnt, coff,
             cand_s, cand_i):
        cid = lax.axis_index("c")
        sid = lax.axis_index("s")
        wid = sid * _NC + cid
        pltpu.sync_copy(xt_hbm, xk)
        qbase = wid * qpw
        inf16 = jnp.full((_LANES,), jnp.inf, jnp.float32)
        zero16 = jnp.zeros((_LANES,), jnp.int32)
        lanes_i = lax.iota(jnp.int32, _LANES)
        lane0 = lanes_i == 0

        def merge16(q, ks, vs):
            for lvl in range(_NLVL):
                bk = outs[q, pl.ds(lvl * _LANES, _LANES)]
                bi = outi[q, pl.ds(lvl * _LANES, _LANES)]
                rk = lax.rev(ks, (0,))
                ri = lax.rev(vs, (0,))
                m = bk <= rk
                lok = jnp.where(m, bk, rk)
                loi = jnp.where(m, bi, ri)
                hik = jnp.where(m, rk, bk)
                hii = jnp.where(m, ri, bi)
                lok, loi = plsc.sort_key_val(lok, loi)
                outs[q, pl.ds(lvl * _LANES, _LANES)] = lok
                outi[q, pl.ds(lvl * _LANES, _LANES)] = loi
                if lvl < _NLVL - 1:
                    ks, vs = plsc.sort_key_val(hik, hii)

        def do_query(q, carry):
            row = qbase + q
            b = row // L
            i = row - b * L
            xoff = b * (3 * L) + i
            qx = plsc.load_gather(xk, [jnp.full((_LANES,), xoff, jnp.int32)])
            qy = plsc.load_gather(xk, [jnp.full((_LANES,), xoff + L,
                                                jnp.int32)])
            qz = plsc.load_gather(xk, [jnp.full((_LANES,), xoff + 2 * L,
                                                jnp.int32)])
            for lvl in range(_NLVL):
                outs[q, pl.ds(lvl * _LANES, _LANES)] = inf16
                outi[q, pl.ds(lvl * _LANES, _LANES)] = zero16

            kbase = b * (3 * L)

            def pass_a(it, bs):
                b1, b2, b3 = bs
                for u in range(_CPB):
                    b0 = (it * _CPB + u) * _LANES
                    kx = xk[pl.ds(kbase + b0, _LANES)]
                    ky = xk[pl.ds(kbase + L + b0, _LANES)]
                    kz = xk[pl.ds(kbase + 2 * L + b0, _LANES)]
                    dx = kx - qx
                    dy = ky - qy
                    dz = kz - qz
                    s = ((dx * dx + dy * dy) + dz * dz) + _EPS
                    sbuf[pl.ds(b0, _LANES)] = s
                    t2 = jnp.maximum(b1, s)
                    b1 = jnp.minimum(b1, s)
                    t3 = jnp.maximum(b2, t2)
                    b2 = jnp.minimum(b2, t2)
                    b3 = jnp.minimum(b3, t3)
                return (b1, b2, b3)

            _, _, b3 = lax.fori_loop(0, nchunk // _CPB, pass_a,
                                     (inf16, inf16, inf16))
            tvec = jnp.full((_LANES,), jnp.max(b3), jnp.float32)

            def pass_b(it, c2):
                for u in range(_CPB):
                    j = it * _CPB + u
                    s = sbuf[pl.ds(j * _LANES, _LANES)]
                    m = s <= tvec
                    cntv = plsc.all_reduce_population_count(m)
                    plsc.store_compressed(ccnt.at[pl.ds(j, _LANES)], cntv,
                                          mask=lane0)
                return c2

            lax.fori_loop(0, nchunk // _CPB, pass_b, 0)

            def pass_p(u, basev):
                c = ccnt[pl.ds(u * _LANES, _LANES)]
                cs = plsc.cumsum(c)
                coff[pl.ds(u * _LANES, _LANES)] = (cs - c) + basev
                return basev + jnp.full((_LANES,), jnp.max(cs), jnp.int32)

            nov = nchunk // _LANES
            totv = lax.fori_loop(0, nov, pass_p,
                                 jnp.zeros((_LANES,), jnp.int32))

            def pass_c(it, c2):
                for u in range(_CPB):
                    j = it * _CPB + u
                    s = sbuf[pl.ds(j * _LANES, _LANES)]
                    m = s <= tvec
                    mi = m.astype(jnp.int32)
                    cum = plsc.cumsum(mi)
                    offv = plsc.load_gather(
                        coff, [jnp.full((_LANES,), j, jnp.int32)])
                    pos = (offv + cum) - 1
                    plsc.store_scatter(cand_s, [pos], s, mask=m)
                    plsc.store_scatter(cand_i, [pos], lanes_i + j * _LANES,
                                       mask=m)
                return c2

            lax.fori_loop(0, nchunk // _CPB, pass_c, 0)

            stot = lax.reduce_max(totv, (0,))
            nv = (stot + (_LANES - 1)) // _LANES

            def fin(v, c2):
                ks = cand_s[pl.ds(v * _LANES, _LANES)]
                vs = cand_i[pl.ds(v * _LANES, _LANES)]
                valid = (lanes_i + v * _LANES) < totv
                ks = jnp.where(valid, ks, jnp.inf)
                ks, vs = plsc.sort_key_val(ks, vs)
                merge16(q, ks, vs)
                return c2

            lax.fori_loop(0, nv, fin, 0)
            return carry

        lax.fori_loop(0, qpw, do_query, 0)
        pltpu.sync_copy(outs, cs_hbm.at[pl.ds(qbase, qpw)])
        pltpu.sync_copy(outi, ci_hbm.at[pl.ds(qbase, qpw)])

    mesh = plsc.VectorSubcoreMesh(core_axis_name="c", subcore_axis_name="s",
                                  num_cores=_NC, num_subcores=_NS)
    f = pl.kernel(
        body,
        out_type=[jax.ShapeDtypeStruct((rows, _KC), jnp.float32),
                  jax.ShapeDtypeStruct((rows, _KC), jnp.int32)],
        mesh=mesh,
        compiler_params=pltpu.CompilerParams(needs_layout_passes=False),
        scratch_types=[
            pltpu.VMEM((B * 3 * L,), jnp.float32),
            pltpu.VMEM((qpw, _KC), jnp.float32),
            pltpu.VMEM((qpw, _KC), jnp.int32),
            pltpu.VMEM((L,), jnp.float32),
            pltpu.VMEM((L // _LANES + _LANES,), jnp.int32),
            pltpu.VMEM((L // _LANES + _LANES,), jnp.int32),
            pltpu.VMEM((L,), jnp.float32),
            pltpu.VMEM((L,), jnp.int32),
        ],
    )
    return f(xt)


def _fix_body(cs_ref, ci_ref, d_ref, i_ref, cm_ref, rm_ref):
    s = cs_ref[...]
    ix = ci_ref[...]
    d = jnp.sqrt(s)
    n = s.shape[-1]
    lane = lax.broadcasted_iota(jnp.int32, s.shape, 1)
    for p in range(_NPASS):
        par = p % 2
        enR = (lane % 2 == par) & (lane < n - 1)
        enL = ((lane + 1) % 2 == par) & (lane >= 1)
        dR = pltpu.roll(d, n - 1, 1)
        ixR = pltpu.roll(ix, n - 1, 1)
        dL = pltpu.roll(d, 1, 1)
        ixL = pltpu.roll(ix, 1, 1)
        swapR = (d == dR) & (ix > ixR) & enR
        swapL = (dL == d) & (ixL > ix) & enL
        ix = jnp.where(swapR, ixR, jnp.where(swapL, ixL, ix))
    d32 = d[:, :_K]
    i32 = ix[:, :_K]
    d_ref[...] = d32
    i_ref[...] = i32
    cm_ref[...] = d32 < jnp.float32(5e7)
    rm_ref[...] = d32 < jnp.float32(5e9)


def _tc_fix(cs, ci):
    rows = cs.shape[0]
    rb = min(512, rows)
    grid = (rows // rb,)
    return pl.pallas_call(
        _fix_body,
        grid=grid,
        in_specs=[
            pl.BlockSpec((rb, _KC), lambda r: (r, 0)),
            pl.BlockSpec((rb, _KC), lambda r: (r, 0)),
        ],
        out_specs=[
            pl.BlockSpec((rb, _K), lambda r: (r, 0)),
            pl.BlockSpec((rb, _K), lambda r: (r, 0)),
            pl.BlockSpec((rb, _K), lambda r: (r, 0)),
            pl.BlockSpec((rb, _K), lambda r: (r, 0)),
        ],
        out_shape=[
            jax.ShapeDtypeStruct((rows, _K), jnp.float32),
            jax.ShapeDtypeStruct((rows, _K), jnp.int32),
            jax.ShapeDtypeStruct((rows, _K), jnp.bool_),
            jax.ShapeDtypeStruct((rows, _K), jnp.bool_),
        ],
    )(cs, ci)


def kernel(X, coord_mask, padding_mask, top_k_neighbors):
    del coord_mask, padding_mask, top_k_neighbors
    B, L = X.shape[0], X.shape[1]
    xt = jnp.transpose(X, (0, 2, 1)).reshape(-1)
    cs, ci = _sc_select(xt, B, L)
    d32, i32, cm, rm = _tc_fix(cs, ci)
    shape = (B, L, _K)
    return (d32.reshape(shape), i32.reshape(shape),
            cm.reshape(shape), rm.reshape(shape))


# speedup vs baseline: 7.6313x; 2.7167x over previous
import functools

import numpy as np

import jax
import jax.numpy as jnp
from jax import lax
from jax.experimental import pallas as pl
from jax.experimental.pallas import tpu as pltpu
from jax.experimental.pallas import tpu_sc as plsc

_NC, _NS, _LANES = 2, 16, 16
_NW = _NC * _NS
_KC = 48
_NLVL = _KC // _LANES
_K = 32
_NPASS = 10
_CPB = 4
_EPS = np.float32(1e-8)


def _sc_select(xt, B, L):
    rows = B * L
    qpw = rows // _NW
    nchunk = L // _LANES

    def body(xt_hbm, cs_hbm, ci_hbm, xk, outs, outi, sbuf, ccnt, coff,
             cand_s, cand_i):
        cid = lax.axis_index("c")
        sid = lax.axis_index("s")
        wid = sid * _NC + cid
        pltpu.sync_copy(xt_hbm, xk)
        qbase = wid * qpw
        inf16 = jnp.full((_LANES,), jnp.inf, jnp.float32)
        zero16 = jnp.zeros((_LANES,), jnp.int32)
        lanes_i = lax.iota(jnp.int32, _LANES)
        lane0 = lanes_i == 0

        def merge16(q, ks, vs):
            for lvl in range(_NLVL):
                bk = outs[q, pl.ds(lvl * _LANES, _LANES)]
                bi = outi[q, pl.ds(lvl * _LANES, _LANES)]
                rk = lax.rev(ks, (0,))
                ri = lax.rev(vs, (0,))
                m = bk <= rk
                lok = jnp.where(m, bk, rk)
                loi = jnp.where(m, bi, ri)
                hik = jnp.where(m, rk, bk)
                hii = jnp.where(m, ri, bi)
                lok, loi = plsc.sort_key_val(lok, loi)
                outs[q, pl.ds(lvl * _LANES, _LANES)] = lok
                outi[q, pl.ds(lvl * _LANES, _LANES)] = loi
                if lvl < _NLVL - 1:
                    ks, vs = plsc.sort_key_val(hik, hii)

        def do_query(q, carry):
            row = qbase + q
            b = row // L
            i = row - b * L
            xoff = b * (3 * L) + i
            qx = plsc.load_gather(xk, [jnp.full((_LANES,), xoff, jnp.int32)])
            qy = plsc.load_gather(xk, [jnp.full((_LANES,), xoff + L,
                                                jnp.int32)])
            qz = plsc.load_gather(xk, [jnp.full((_LANES,), xoff + 2 * L,
                                                jnp.int32)])
            for lvl in range(_NLVL):
                outs[q, pl.ds(lvl * _LANES, _LANES)] = inf16
                outi[q, pl.ds(lvl * _LANES, _LANES)] = zero16

            kbase = b * (3 * L)

            @plsc.parallel_loop(0, nchunk, step=_CPB, unroll=2,
                                carry=tuple((inf16, inf16, inf16)
                                            for _ in range(_CPB)))
            def pass_a(j, trk):
                newtrk = []
                for u in range(_CPB):
                    b0 = (j + u) * _LANES
                    kx = xk[pl.ds(kbase + b0, _LANES)]
                    ky = xk[pl.ds(kbase + L + b0, _LANES)]
                    kz = xk[pl.ds(kbase + 2 * L + b0, _LANES)]
                    dx = kx - qx
                    dy = ky - qy
                    dz = kz - qz
                    s = ((dx * dx + dy * dy) + dz * dz) + _EPS
                    sbuf[pl.ds(b0, _LANES)] = s
                    b1, b2, b3 = trk[u]
                    t2 = jnp.maximum(b1, s)
                    b1 = jnp.minimum(b1, s)
                    t3 = jnp.maximum(b2, t2)
                    b2 = jnp.minimum(b2, t2)
                    b3 = jnp.minimum(b3, t3)
                    newtrk.append((b1, b2, b3))
                return tuple(newtrk)

            b1, b2, b3 = pass_a[0]
            for u in range(1, _CPB):
                for sv in pass_a[u]:
                    t2 = jnp.maximum(b1, sv)
                    b1 = jnp.minimum(b1, sv)
                    t3 = jnp.maximum(b2, t2)
                    b2 = jnp.minimum(b2, t2)
                    b3 = jnp.minimum(b3, t3)
            tvec = jnp.full((_LANES,), jnp.max(b3), jnp.float32)

            @plsc.parallel_loop(0, nchunk, step=1, unroll=_CPB)
            def pass_b(j):
                s = sbuf[pl.ds(j * _LANES, _LANES)]
                m = s <= tvec
                cntv = plsc.all_reduce_population_count(m)
                plsc.store_compressed(ccnt.at[pl.ds(j, _LANES)], cntv,
                                      mask=lane0)

            def pass_p(u, basev):
                c = ccnt[pl.ds(u * _LANES, _LANES)]
                cs = plsc.cumsum(c)
                coff[pl.ds(u * _LANES, _LANES)] = (cs - c) + basev
                return basev + jnp.full((_LANES,), jnp.max(cs), jnp.int32)

            nov = nchunk // _LANES
            totv = lax.fori_loop(0, nov, pass_p,
                                 jnp.zeros((_LANES,), jnp.int32))

            @plsc.parallel_loop(0, nchunk, step=1, unroll=_CPB)
            def pass_c(j):
                s = sbuf[pl.ds(j * _LANES, _LANES)]
                m = s <= tvec
                mi = m.astype(jnp.int32)
                cum = plsc.cumsum(mi)
                offv = plsc.load_gather(
                    coff, [jnp.full((_LANES,), j, jnp.int32)])
                pos = (offv + cum) - 1
                plsc.store_scatter(cand_s, [pos], s, mask=m)
                plsc.store_scatter(cand_i, [pos], lanes_i + j * _LANES,
                                   mask=m)

            stot = lax.reduce_max(totv, (0,))
            nv = (stot + (_LANES - 1)) // _LANES

            def fin(v, c2):
                ks = cand_s[pl.ds(v * _LANES, _LANES)]
                vs = cand_i[pl.ds(v * _LANES, _LANES)]
                valid = (lanes_i + v * _LANES) < totv
                ks = jnp.where(valid, ks, jnp.inf)
                ks, vs = plsc.sort_key_val(ks, vs)
                merge16(q, ks, vs)
                return c2

            lax.fori_loop(0, nv, fin, 0)
            return carry

        lax.fori_loop(0, qpw, do_query, 0)
        pltpu.sync_copy(outs, cs_hbm.at[pl.ds(qbase, qpw)])
        pltpu.sync_copy(outi, ci_hbm.at[pl.ds(qbase, qpw)])

    mesh = plsc.VectorSubcoreMesh(core_axis_name="c", subcore_axis_name="s",
                                  num_cores=_NC, num_subcores=_NS)
    f = pl.kernel(
        body,
        out_type=[jax.ShapeDtypeStruct((rows, _KC), jnp.float32),
                  jax.ShapeDtypeStruct((rows, _KC), jnp.int32)],
        mesh=mesh,
        compiler_params=pltpu.CompilerParams(needs_layout_passes=False),
        scratch_types=[
            pltpu.VMEM((B * 3 * L,), jnp.float32),
            pltpu.VMEM((qpw, _KC), jnp.float32),
            pltpu.VMEM((qpw, _KC), jnp.int32),
            pltpu.VMEM((L,), jnp.float32),
            pltpu.VMEM((L // _LANES + _LANES,), jnp.int32),
            pltpu.VMEM((L // _LANES + _LANES,), jnp.int32),
            pltpu.VMEM((L,), jnp.float32),
            pltpu.VMEM((L,), jnp.int32),
        ],
    )
    return f(xt)


def _fix_body(cs_ref, ci_ref, d_ref, i_ref, cm_ref, rm_ref):
    s = cs_ref[...]
    ix = ci_ref[...]
    d = jnp.sqrt(s)
    n = s.shape[-1]
    lane = lax.broadcasted_iota(jnp.int32, s.shape, 1)
    for p in range(_NPASS):
        par = p % 2
        enR = (lane % 2 == par) & (lane < n - 1)
        enL = ((lane + 1) % 2 == par) & (lane >= 1)
        dR = pltpu.roll(d, n - 1, 1)
        ixR = pltpu.roll(ix, n - 1, 1)
        dL = pltpu.roll(d, 1, 1)
        ixL = pltpu.roll(ix, 1, 1)
        swapR = (d == dR) & (ix > ixR) & enR
        swapL = (dL == d) & (ixL > ix) & enL
        ix = jnp.where(swapR, ixR, jnp.where(swapL, ixL, ix))
    d32 = d[:, :_K]
    i32 = ix[:, :_K]
    d_ref[...] = d32
    i_ref[...] = i32
    cm_ref[...] = d32 < jnp.float32(5e7)
    rm_ref[...] = d32 < jnp.float32(5e9)


def _tc_fix(cs, ci):
    rows = cs.shape[0]
    rb = min(512, rows)
    grid = (rows // rb,)
    return pl.pallas_call(
        _fix_body,
        grid=grid,
        in_specs=[
            pl.BlockSpec((rb, _KC), lambda r: (r, 0)),
            pl.BlockSpec((rb, _KC), lambda r: (r, 0)),
        ],
        out_specs=[
            pl.BlockSpec((rb, _K), lambda r: (r, 0)),
            pl.BlockSpec((rb, _K), lambda r: (r, 0)),
            pl.BlockSpec((rb, _K), lambda r: (r, 0)),
            pl.BlockSpec((rb, _K), lambda r: (r, 0)),
        ],
        out_shape=[
            jax.ShapeDtypeStruct((rows, _K), jnp.float32),
            jax.ShapeDtypeStruct((rows, _K), jnp.int32),
            jax.ShapeDtypeStruct((rows, _K), jnp.bool_),
            jax.ShapeDtypeStruct((rows, _K), jnp.bool_),
        ],
    )(cs, ci)


def kernel(X, coord_mask, padding_mask, top_k_neighbors):
    del coord_mask, padding_mask, top_k_neighbors
    B, L = X.shape[0], X.shape[1]
    xt = jnp.transpose(X, (0, 2, 1)).reshape(-1)
    cs, ci = _sc_select(xt, B, L)
    d32, i32, cm, rm = _tc_fix(cs, ci)
    shape = (B, L, _K)
    return (d32.reshape(shape), i32.reshape(shape),
            cm.reshape(shape), rm.reshape(shape))


# speedup vs baseline: 8.2334x; 1.0789x over previous
import functools

import numpy as np

import jax
import jax.numpy as jnp
from jax import lax
from jax.experimental import pallas as pl
from jax.experimental.pallas import tpu as pltpu
from jax.experimental.pallas import tpu_sc as plsc

_NC, _NS, _LANES = 2, 16, 16
_NW = _NC * _NS
_KC = 48
_NLVL = _KC // _LANES
_K = 32
_NPASS = 6
_CPB = 4
_EPS = np.float32(1e-8)


def _sc_select(xt, B, L):
    rows = B * L
    qpw = rows // _NW
    nchunk = L // _LANES

    def body(xt_hbm, cs_hbm, ci_hbm, xk, outs, outi, sbuf, ccnt, coff,
             cand_s, cand_i):
        cid = lax.axis_index("c")
        sid = lax.axis_index("s")
        wid = sid * _NC + cid
        pltpu.sync_copy(xt_hbm, xk)
        qbase = wid * qpw
        inf16 = jnp.full((_LANES,), jnp.inf, jnp.float32)
        zero16 = jnp.zeros((_LANES,), jnp.int32)
        lanes_i = lax.iota(jnp.int32, _LANES)
        lane0 = lanes_i == 0

        def merge16(q, ks, vs):
            for lvl in range(_NLVL):
                bk = outs[q, pl.ds(lvl * _LANES, _LANES)]
                bi = outi[q, pl.ds(lvl * _LANES, _LANES)]
                rk = lax.rev(ks, (0,))
                ri = lax.rev(vs, (0,))
                m = bk <= rk
                lok = jnp.where(m, bk, rk)
                loi = jnp.where(m, bi, ri)
                hik = jnp.where(m, rk, bk)
                hii = jnp.where(m, ri, bi)
                lok, loi = plsc.sort_key_val(lok, loi)
                outs[q, pl.ds(lvl * _LANES, _LANES)] = lok
                outi[q, pl.ds(lvl * _LANES, _LANES)] = loi
                if lvl < _NLVL - 1:
                    ks, vs = plsc.sort_key_val(hik, hii)

        def do_query(q, carry):
            row = qbase + q
            b = row // L
            i = row - b * L
            xoff = b * (3 * L) + i
            qx = plsc.load_gather(xk, [jnp.full((_LANES,), xoff, jnp.int32)])
            qy = plsc.load_gather(xk, [jnp.full((_LANES,), xoff + L,
                                                jnp.int32)])
            qz = plsc.load_gather(xk, [jnp.full((_LANES,), xoff + 2 * L,
                                                jnp.int32)])
            for lvl in range(_NLVL):
                outs[q, pl.ds(lvl * _LANES, _LANES)] = inf16
                outi[q, pl.ds(lvl * _LANES, _LANES)] = zero16

            kbase = b * (3 * L)

            @plsc.parallel_loop(0, nchunk, step=_CPB, unroll=2,
                                carry=tuple((inf16, inf16, inf16)
                                            for _ in range(_CPB)))
            def pass_a(j, trk):
                newtrk = []
                for u in range(_CPB):
                    b0 = (j + u) * _LANES
                    kx = xk[pl.ds(kbase + b0, _LANES)]
                    ky = xk[pl.ds(kbase + L + b0, _LANES)]
                    kz = xk[pl.ds(kbase + 2 * L + b0, _LANES)]
                    dx = kx - qx
                    dy = ky - qy
                    dz = kz - qz
                    s = ((dx * dx + dy * dy) + dz * dz) + _EPS
                    sbuf[pl.ds(b0, _LANES)] = s
                    b1, b2, b3 = trk[u]
                    t2 = jnp.maximum(b1, s)
                    b1 = jnp.minimum(b1, s)
                    t3 = jnp.maximum(b2, t2)
                    b2 = jnp.minimum(b2, t2)
                    b3 = jnp.minimum(b3, t3)
                    newtrk.append((b1, b2, b3))
                return tuple(newtrk)

            b1, b2, b3 = pass_a[0]
            for u in range(1, _CPB):
                for sv in pass_a[u]:
                    t2 = jnp.maximum(b1, sv)
                    b1 = jnp.minimum(b1, sv)
                    t3 = jnp.maximum(b2, t2)
                    b2 = jnp.minimum(b2, t2)
                    b3 = jnp.minimum(b3, t3)
            tvec = jnp.full((_LANES,), jnp.max(b3), jnp.float32)

            @plsc.parallel_loop(0, nchunk, step=1, unroll=8)
            def pass_b(j):
                s = sbuf[pl.ds(j * _LANES, _LANES)]
                m = s <= tvec
                cntv = plsc.all_reduce_population_count(m)
                plsc.store_compressed(ccnt.at[pl.ds(j, _LANES)], cntv,
                                      mask=lane0)

            def pass_p(u, basev):
                c = ccnt[pl.ds(u * _LANES, _LANES)]
                cs = plsc.cumsum(c)
                coff[pl.ds(u * _LANES, _LANES)] = (cs - c) + basev
                return basev + jnp.full((_LANES,), jnp.max(cs), jnp.int32)

            nov = nchunk // _LANES
            totv = lax.fori_loop(0, nov, pass_p,
                                 jnp.zeros((_LANES,), jnp.int32))

            @plsc.parallel_loop(0, nchunk, step=1, unroll=_CPB)
            def pass_c(j):
                s = sbuf[pl.ds(j * _LANES, _LANES)]
                m = s <= tvec
                mi = m.astype(jnp.int32)
                cum = plsc.cumsum(mi)
                offv = plsc.load_gather(
                    coff, [jnp.full((_LANES,), j, jnp.int32)])
                pos = (offv + cum) - 1
                plsc.store_scatter(cand_s, [pos], s, mask=m)
                plsc.store_scatter(cand_i, [pos], lanes_i + j * _LANES,
                                   mask=m)

            stot = lax.reduce_max(totv, (0,))
            nv = (stot + (_LANES - 1)) // _LANES

            def fin(v, c2):
                ks = cand_s[pl.ds(v * _LANES, _LANES)]
                vs = cand_i[pl.ds(v * _LANES, _LANES)]
                valid = (lanes_i + v * _LANES) < totv
                ks = jnp.where(valid, ks, jnp.inf)
                ks, vs = plsc.sort_key_val(ks, vs)
                merge16(q, ks, vs)
                return c2

            lax.fori_loop(0, nv, fin, 0)
            return carry

        lax.fori_loop(0, qpw, do_query, 0)
        pltpu.sync_copy(outs, cs_hbm.at[pl.ds(qbase, qpw)])
        pltpu.sync_copy(outi, ci_hbm.at[pl.ds(qbase, qpw)])

    mesh = plsc.VectorSubcoreMesh(core_axis_name="c", subcore_axis_name="s",
                                  num_cores=_NC, num_subcores=_NS)
    f = pl.kernel(
        body,
        out_type=[jax.ShapeDtypeStruct((rows, _KC), jnp.float32),
                  jax.ShapeDtypeStruct((rows, _KC), jnp.int32)],
        mesh=mesh,
        compiler_params=pltpu.CompilerParams(needs_layout_passes=False),
        scratch_types=[
            pltpu.VMEM((B * 3 * L,), jnp.float32),
            pltpu.VMEM((qpw, _KC), jnp.float32),
            pltpu.VMEM((qpw, _KC), jnp.int32),
            pltpu.VMEM((L,), jnp.float32),
            pltpu.VMEM((L // _LANES + _LANES,), jnp.int32),
            pltpu.VMEM((L // _LANES + _LANES,), jnp.int32),
            pltpu.VMEM((L,), jnp.float32),
            pltpu.VMEM((L,), jnp.int32),
        ],
    )
    return f(xt)


def _fix_body(cs_ref, ci_ref, d_ref, i_ref, cm_ref, rm_ref):
    s = cs_ref[...]
    ix = ci_ref[...]
    d = jnp.sqrt(s)
    n = s.shape[-1]
    lane = lax.broadcasted_iota(jnp.int32, s.shape, 1)
    for p in range(_NPASS):
        par = p % 2
        enR = (lane % 2 == par) & (lane < n - 1)
        enL = ((lane + 1) % 2 == par) & (lane >= 1)
        dR = pltpu.roll(d, n - 1, 1)
        ixR = pltpu.roll(ix, n - 1, 1)
        dL = pltpu.roll(d, 1, 1)
        ixL = pltpu.roll(ix, 1, 1)
        swapR = (d == dR) & (ix > ixR) & enR
        swapL = (dL == d) & (ixL > ix) & enL
        ix = jnp.where(swapR, ixR, jnp.where(swapL, ixL, ix))
    d32 = d[:, :_K]
    i32 = ix[:, :_K]
    d_ref[...] = d32
    i_ref[...] = i32
    cm_ref[...] = d32 < jnp.float32(5e7)
    rm_ref[...] = d32 < jnp.float32(5e9)


def _tc_fix(cs, ci):
    rows = cs.shape[0]
    rb = min(512, rows)
    grid = (rows // rb,)
    return pl.pallas_call(
        _fix_body,
        grid=grid,
        in_specs=[
            pl.BlockSpec((rb, _KC), lambda r: (r, 0)),
            pl.BlockSpec((rb, _KC), lambda r: (r, 0)),
        ],
        out_specs=[
            pl.BlockSpec((rb, _K), lambda r: (r, 0)),
            pl.BlockSpec((rb, _K), lambda r: (r, 0)),
            pl.BlockSpec((rb, _K), lambda r: (r, 0)),
            pl.BlockSpec((rb, _K), lambda r: (r, 0)),
        ],
        out_shape=[
            jax.ShapeDtypeStruct((rows, _K), jnp.float32),
            jax.ShapeDtypeStruct((rows, _K), jnp.int32),
            jax.ShapeDtypeStruct((rows, _K), jnp.bool_),
            jax.ShapeDtypeStruct((rows, _K), jnp.bool_),
        ],
    )(cs, ci)


def kernel(X, coord_mask, padding_mask, top_k_neighbors):
    del coord_mask, padding_mask, top_k_neighbors
    B, L = X.shape[0], X.shape[1]
    xt = jnp.transpose(X, (0, 2, 1)).reshape(-1)
    cs, ci = _sc_select(xt, B, L)
    d32, i32, cm, rm = _tc_fix(cs, ci)
    shape = (B, L, _K)
    return (d32.reshape(shape), i32.reshape(shape),
            cm.reshape(shape), rm.reshape(shape))


# speedup vs baseline: 8.2346x; 1.0001x over previous
import functools

import numpy as np

import jax
import jax.numpy as jnp
from jax import lax
from jax.experimental import pallas as pl
from jax.experimental.pallas import tpu as pltpu
from jax.experimental.pallas import tpu_sc as plsc

_NC, _NS, _LANES = 2, 16, 16
_NW = _NC * _NS
_KC = 48
_NLVL = _KC // _LANES
_K = 32
_NPASS = 6
_CPB = 4
_EPS = np.float32(1e-8)


def _sc_select(xt, B, L):
    rows = B * L
    qpw = rows // _NW
    nchunk = L // _LANES
    _CC = nchunk + _LANES

    def body(xt_hbm, cs_hbm, ci_hbm, xk, outs, outi, sbuf, ccnt, coff,
             cand_s, cand_i):
        cid = lax.axis_index("c")
        sid = lax.axis_index("s")
        wid = sid * _NC + cid
        pltpu.sync_copy(xt_hbm, xk)
        qbase = wid * qpw
        inf16 = jnp.full((_LANES,), jnp.inf, jnp.float32)
        zero16 = jnp.zeros((_LANES,), jnp.int32)
        lanes_i = lax.iota(jnp.int32, _LANES)
        lane0 = lanes_i == 0

        def merge16(q, ks, vs):
            for lvl in range(_NLVL):
                bk = outs[q, pl.ds(lvl * _LANES, _LANES)]
                bi = outi[q, pl.ds(lvl * _LANES, _LANES)]
                rk = lax.rev(ks, (0,))
                ri = lax.rev(vs, (0,))
                m = bk <= rk
                lok = jnp.where(m, bk, rk)
                loi = jnp.where(m, bi, ri)
                hik = jnp.where(m, rk, bk)
                hii = jnp.where(m, ri, bi)
                lok, loi = plsc.sort_key_val(lok, loi)
                outs[q, pl.ds(lvl * _LANES, _LANES)] = lok
                outi[q, pl.ds(lvl * _LANES, _LANES)] = loi
                if lvl < _NLVL - 1:
                    ks, vs = plsc.sort_key_val(hik, hii)

        def do_query(p, q):
            soff = p * L
            cbase = p * _CC
            row = qbase + q
            b = row // L
            i = row - b * L
            xoff = b * (3 * L) + i
            qx = plsc.load_gather(xk, [jnp.full((_LANES,), xoff, jnp.int32)])
            qy = plsc.load_gather(xk, [jnp.full((_LANES,), xoff + L,
                                                jnp.int32)])
            qz = plsc.load_gather(xk, [jnp.full((_LANES,), xoff + 2 * L,
                                                jnp.int32)])
            for lvl in range(_NLVL):
                outs[q, pl.ds(lvl * _LANES, _LANES)] = inf16
                outi[q, pl.ds(lvl * _LANES, _LANES)] = zero16

            kbase = b * (3 * L)

            @plsc.parallel_loop(0, nchunk, step=_CPB, unroll=2,
                                carry=tuple((inf16, inf16, inf16)
                                            for _ in range(_CPB)))
            def pass_a(j, trk):
                newtrk = []
                for u in range(_CPB):
                    b0 = (j + u) * _LANES
                    kx = xk[pl.ds(kbase + b0, _LANES)]
                    ky = xk[pl.ds(kbase + L + b0, _LANES)]
                    kz = xk[pl.ds(kbase + 2 * L + b0, _LANES)]
                    dx = kx - qx
                    dy = ky - qy
                    dz = kz - qz
                    s = ((dx * dx + dy * dy) + dz * dz) + _EPS
                    sbuf[pl.ds(soff + b0, _LANES)] = s
                    b1, b2, b3 = trk[u]
                    t2 = jnp.maximum(b1, s)
                    b1 = jnp.minimum(b1, s)
                    t3 = jnp.maximum(b2, t2)
                    b2 = jnp.minimum(b2, t2)
                    b3 = jnp.minimum(b3, t3)
                    newtrk.append((b1, b2, b3))
                return tuple(newtrk)

            b1, b2, b3 = pass_a[0]
            for u in range(1, _CPB):
                for sv in pass_a[u]:
                    t2 = jnp.maximum(b1, sv)
                    b1 = jnp.minimum(b1, sv)
                    t3 = jnp.maximum(b2, t2)
                    b2 = jnp.minimum(b2, t2)
                    b3 = jnp.minimum(b3, t3)
            tvec = jnp.full((_LANES,), jnp.max(b3), jnp.float32)

            @plsc.parallel_loop(0, nchunk, step=1, unroll=8)
            def pass_b(j):
                s = sbuf[pl.ds(soff + j * _LANES, _LANES)]
                m = s <= tvec
                cntv = plsc.all_reduce_population_count(m)
                plsc.store_compressed(ccnt.at[pl.ds(cbase + j, _LANES)], cntv,
                                      mask=lane0)

            def pass_p(u, basev):
                c = ccnt[pl.ds(cbase + u * _LANES, _LANES)]
                cs = plsc.cumsum(c)
                coff[pl.ds(cbase + u * _LANES, _LANES)] = (cs - c) + basev
                return basev + jnp.full((_LANES,), jnp.max(cs), jnp.int32)

            nov = nchunk // _LANES
            totv = lax.fori_loop(0, nov, pass_p,
                                 jnp.zeros((_LANES,), jnp.int32))

            @plsc.parallel_loop(0, nchunk, step=1, unroll=_CPB)
            def pass_c(j):
                s = sbuf[pl.ds(soff + j * _LANES, _LANES)]
                m = s <= tvec
                mi = m.astype(jnp.int32)
                cum = plsc.cumsum(mi)
                offv = plsc.load_gather(
                    coff, [jnp.full((_LANES,), cbase + j, jnp.int32)])
                pos = ((offv + cum) - 1) + soff
                plsc.store_scatter(cand_s, [pos], s, mask=m)
                plsc.store_scatter(cand_i, [pos], lanes_i + j * _LANES,
                                   mask=m)

            stot = lax.reduce_max(totv, (0,))
            nv = (stot + (_LANES - 1)) // _LANES

            def fin(v, c2):
                ks = cand_s[pl.ds(soff + v * _LANES, _LANES)]
                vs = cand_i[pl.ds(soff + v * _LANES, _LANES)]
                valid = (lanes_i + v * _LANES) < totv
                ks = jnp.where(valid, ks, jnp.inf)
                ks, vs = plsc.sort_key_val(ks, vs)
                merge16(q, ks, vs)
                return c2

            lax.fori_loop(0, nv, fin, 0)

        def do_group(g, carry):
            @plsc.parallel_loop(0, 2, step=1, unroll=2)
            def _pair(p):
                do_query(p, g * 2 + p)
            return carry

        lax.fori_loop(0, qpw // 2, do_group, 0)
        pltpu.sync_copy(outs, cs_hbm.at[pl.ds(qbase, qpw)])
        pltpu.sync_copy(outi, ci_hbm.at[pl.ds(qbase, qpw)])

    mesh = plsc.VectorSubcoreMesh(core_axis_name="c", subcore_axis_name="s",
                                  num_cores=_NC, num_subcores=_NS)
    f = pl.kernel(
        body,
        out_type=[jax.ShapeDtypeStruct((rows, _KC), jnp.float32),
                  jax.ShapeDtypeStruct((rows, _KC), jnp.int32)],
        mesh=mesh,
        compiler_params=pltpu.CompilerParams(needs_layout_passes=False),
        scratch_types=[
            pltpu.VMEM((B * 3 * L,), jnp.float32),
            pltpu.VMEM((qpw, _KC), jnp.float32),
            pltpu.VMEM((qpw, _KC), jnp.int32),
            pltpu.VMEM((2 * L,), jnp.float32),
            pltpu.VMEM((2 * _CC,), jnp.int32),
            pltpu.VMEM((2 * _CC,), jnp.int32),
            pltpu.VMEM((2 * L,), jnp.float32),
            pltpu.VMEM((2 * L,), jnp.int32),
        ],
    )
    return f(xt)


def _fix_body(cs_ref, ci_ref, d_ref, i_ref, cm_ref, rm_ref):
    s = cs_ref[...]
    ix = ci_ref[...]
    d = jnp.sqrt(s)
    n = s.shape[-1]
    lane = lax.broadcasted_iota(jnp.int32, s.shape, 1)
    for p in range(_NPASS):
        par = p % 2
        enR = (lane % 2 == par) & (lane < n - 1)
        enL = ((lane + 1) % 2 == par) & (lane >= 1)
        dR = pltpu.roll(d, n - 1, 1)
        ixR = pltpu.roll(ix, n - 1, 1)
        dL = pltpu.roll(d, 1, 1)
        ixL = pltpu.roll(ix, 1, 1)
        swapR = (d == dR) & (ix > ixR) & enR
        swapL = (dL == d) & (ixL > ix) & enL
        ix = jnp.where(swapR, ixR, jnp.where(swapL, ixL, ix))
    d32 = d[:, :_K]
    i32 = ix[:, :_K]
    d_ref[...] = d32
    i_ref[...] = i32
    cm_ref[...] = d32 < jnp.float32(5e7)
    rm_ref[...] = d32 < jnp.float32(5e9)


def _tc_fix(cs, ci):
    rows = cs.shape[0]
    rb = min(512, rows)
    grid = (rows // rb,)
    return pl.pallas_call(
        _fix_body,
        grid=grid,
        in_specs=[
            pl.BlockSpec((rb, _KC), lambda r: (r, 0)),
            pl.BlockSpec((rb, _KC), lambda r: (r, 0)),
        ],
        out_specs=[
            pl.BlockSpec((rb, _K), lambda r: (r, 0)),
            pl.BlockSpec((rb, _K), lambda r: (r, 0)),
            pl.BlockSpec((rb, _K), lambda r: (r, 0)),
            pl.BlockSpec((rb, _K), lambda r: (r, 0)),
        ],
        out_shape=[
            jax.ShapeDtypeStruct((rows, _K), jnp.float32),
            jax.ShapeDtypeStruct((rows, _K), jnp.int32),
            jax.ShapeDtypeStruct((rows, _K), jnp.bool_),
            jax.ShapeDtypeStruct((rows, _K), jnp.bool_),
        ],
    )(cs, ci)


def kernel(X, coord_mask, padding_mask, top_k_neighbors):
    del coord_mask, padding_mask, top_k_neighbors
    B, L = X.shape[0], X.shape[1]
    xt = jnp.transpose(X, (0, 2, 1)).reshape(-1)
    cs, ci = _sc_select(xt, B, L)
    d32, i32, cm, rm = _tc_fix(cs, ci)
    shape = (B, L, _K)
    return (d32.reshape(shape), i32.reshape(shape),
            cm.reshape(shape), rm.reshape(shape))
